# baseline (speedup 1.0000x reference)
import functools

import jax
import jax.numpy as jnp
from jax import lax
from jax.experimental import pallas as pl
from jax.experimental.pallas import tpu as pltpu
from jax.experimental.pallas import tpu_sc as plsc

C = 19
TH = 256
W = 512

SC_ROWS_PER_W = 16
SC_CHUNK = 4


def _tree_max(vals):
    while len(vals) > 1:
        nxt = [jnp.maximum(a, b) for a, b in zip(vals[0::2], vals[1::2])]
        if len(vals) % 2:
            nxt.append(vals[-1])
        vals = nxt
    return vals[0]


def _tc_count_kernel(out_ref, tgt_ref, cnt_ref, acc_ref, *, nb, nh):
    b = pl.program_id(0)
    h = pl.program_id(1)

    @pl.when(jnp.logical_and(b == 0, h == 0))
    def _():
        acc_ref[...] = jnp.zeros_like(acc_ref)

    one = jnp.float32(1.0)
    zero = jnp.float32(0.0)
    zvreg = jnp.zeros((8, 128), jnp.float32)
    packed = [zvreg] * C

    for i in range(TH // 8):
        rs = slice(i * 8, (i + 1) * 8)
        for j in range(W // 128):
            cs = slice(j * 128, (j + 1) * 128)
            xt = [out_ref[0, c, rs, cs] for c in range(C)]
            tt = tgt_ref[0, rs, cs]
            m = _tree_max(xt)
            for c in range(C):
                mask_x = xt[c] == m
                mask_t = tt == c
                hit = jnp.where(mask_x, jnp.float32(65793.0), one)
                miss = jnp.where(mask_x, jnp.float32(256.0), zero)
                packed[c] = packed[c] + jnp.where(mask_t, hit, miss)

    for c in range(C):
        a = packed[c]
        tp_f = jnp.floor(a * (1.0 / 65536.0))
        r = a - tp_f * 65536.0
        hp_f = jnp.floor(r * (1.0 / 256.0))
        ht_f = r - hp_f * 256.0
        acc_ref[0, c] += ht_f
        acc_ref[1, c] += hp_f
        acc_ref[2, c] += tp_f

    @pl.when(jnp.logical_and(b == nb - 1, h == nh - 1))
    def _():
        cnt_ref[...] = acc_ref[...]


def _sc_count_kernel(x_hbm, t_hbm, out_hbm, xbuf, tbuf, obuf):
    nc = 2
    wid = lax.axis_index("s") * nc + lax.axis_index("c")
    row0 = wid * SC_ROWS_PER_W

    accs = [jnp.zeros((16,), jnp.int32) for _ in range(C)]
    t_only = jnp.int32(1)
    x_only = jnp.int32(1 << 10)
    both = jnp.int32(1 + (1 << 10) + (1 << 20))
    zero_i = jnp.int32(0)

    for ch in range(SC_ROWS_PER_W // SC_CHUNK):
        r0 = row0 + ch * SC_CHUNK
        pltpu.sync_copy(x_hbm.at[:, pl.ds(r0, SC_CHUNK), :], xbuf)
        pltpu.sync_copy(t_hbm.at[pl.ds(r0, SC_CHUNK), :], tbuf)
        for r in range(SC_CHUNK):
            def body(g, carry):
                col = g * 16
                tt = tbuf[r, pl.ds(col, 16)]
                xt = [xbuf[c, r, pl.ds(col, 16)] for c in range(C)]
                m = _tree_max(xt)
                new = []
                for c in range(C):
                    mask_x = xt[c] == m
                    mask_t = tt == c
                    hit = jnp.where(mask_x, both, t_only)
                    miss = jnp.where(mask_x, x_only, zero_i)
                    new.append(carry[c] + jnp.where(mask_t, hit, miss))
                return tuple(new)

            accs = list(lax.fori_loop(0, W // 16, body, tuple(accs)))

    mask10 = jnp.int32(1023)
    for c in range(C):
        a = accs[c]
        obuf[0, c] = a & mask10
        obuf[1, c] = (a >> 10) & mask10
        obuf[2, c] = a >> 20
    pltpu.sync_copy(obuf, out_hbm.at[wid])


def _finish_kernel(tc_ref, sc_ref, loss_ref, *, n_total):
    tc_sums = jnp.sum(tc_ref[...], axis=(2, 3))
    sc_sums = jnp.sum(sc_ref[...].astype(jnp.float32), axis=(0, 3))
    sums = tc_sums + sc_sums
    ht_a = sums[0]
    hp_a = sums[1]
    tp_a = sums[2]
    n = jnp.float32(n_total)
    sens = (tp_a + 1.0) / (hp_a + 1.0)
    spec = (n - ht_a - hp_a + tp_a + 1.0) / (n - hp_a + 1.0)
    loss = 1.0 - jnp.mean(0.5 * sens + 0.5 * spec)
    loss_ref[...] = jnp.reshape(loss, (1, 1))


def kernel(output, target):
    B, num_classes, H, _ = output.shape
    assert num_classes == C
    nb_tc = B - 1
    nh = H // TH
    n_total = B * H * W

    tc_counts = pl.pallas_call(
        functools.partial(_tc_count_kernel, nb=nb_tc, nh=nh),
        grid=(nb_tc, nh),
        in_specs=[
            pl.BlockSpec((1, C, TH, W), lambda b, h: (b, 0, h, 0)),
            pl.BlockSpec((1, TH, W), lambda b, h: (b, h, 0)),
        ],
        out_specs=pl.BlockSpec((3, C, 8, 128), lambda b, h: (0, 0, 0, 0)),
        out_shape=jax.ShapeDtypeStruct((3, C, 8, 128), jnp.float32),
        scratch_shapes=[pltpu.VMEM((3, C, 8, 128), jnp.float32)],
    )(output[:nb_tc], target[:nb_tc])

    mesh = plsc.VectorSubcoreMesh(core_axis_name="c", subcore_axis_name="s")
    sc_counts = functools.partial(
        pl.kernel,
        mesh=mesh,
        out_type=jax.ShapeDtypeStruct((32, 3, C, 16), jnp.int32),
        scratch_types=[
            pltpu.VMEM((C, SC_CHUNK, W), jnp.float32),
            pltpu.VMEM((SC_CHUNK, W), jnp.int32),
            pltpu.VMEM((3, C, 16), jnp.int32),
        ],
    )(_sc_count_kernel)(output[B - 1], target[B - 1])

    loss = pl.pallas_call(
        functools.partial(_finish_kernel, n_total=n_total),
        out_shape=jax.ShapeDtypeStruct((1, 1), jnp.float32),
    )(tc_counts, sc_counts)
    return loss[0, 0]


# speedup vs baseline: 3.2686x; 3.2686x over previous
import functools

import jax
import jax.numpy as jnp
from jax.experimental import pallas as pl
from jax.experimental.pallas import tpu as pltpu

C = 19
TH = 256


def _tree_max(vals):
    while len(vals) > 1:
        nxt = [jnp.maximum(a, b) for a, b in zip(vals[0::2], vals[1::2])]
        if len(vals) % 2:
            nxt.append(vals[-1])
        vals = nxt
    return vals[0]


def _loss_kernel(out_ref, tgt_ref, loss_ref, acc_ref, *, nb, nh, n_total):
    b = pl.program_id(0)
    h = pl.program_id(1)

    @pl.when(jnp.logical_and(b == 0, h == 0))
    def _():
        acc_ref[...] = jnp.zeros_like(acc_ref)

    one = jnp.float32(1.0)
    zero = jnp.float32(0.0)
    zvreg = jnp.zeros((8, 128), jnp.float32)
    packed = [zvreg] * C

    for i in range(TH // 8):
        rs = slice(i * 8, (i + 1) * 8)
        for j in range(512 // 128):
            cs = slice(j * 128, (j + 1) * 128)
            xt = [out_ref[0, c, rs, cs] for c in range(C)]
            tt = tgt_ref[0, rs, cs]
            m = _tree_max(xt)
            for c in range(C):
                mask_x = xt[c] == m
                mask_t = tt == c
                hit = jnp.where(mask_x, jnp.float32(65793.0), one)
                miss = jnp.where(mask_x, jnp.float32(256.0), zero)
                packed[c] = packed[c] + jnp.where(mask_t, hit, miss)

    for c in range(C):
        a = packed[c]
        tp_f = jnp.floor(a * (1.0 / 65536.0))
        r = a - tp_f * 65536.0
        hp_f = jnp.floor(r * (1.0 / 256.0))
        ht_f = r - hp_f * 256.0
        acc_ref[0, c] += ht_f
        acc_ref[1, c] += hp_f
        acc_ref[2, c] += tp_f

    @pl.when(jnp.logical_and(b == nb - 1, h == nh - 1))
    def _():
        sums = jnp.sum(acc_ref[...], axis=(2, 3))
        ht_a = sums[0]
        hp_a = sums[1]
        tp_a = sums[2]
        n = jnp.float32(n_total)
        sens = (tp_a + 1.0) / (hp_a + 1.0)
        spec = (n - ht_a - hp_a + tp_a + 1.0) / (n - hp_a + 1.0)
        loss = 1.0 - jnp.mean(0.5 * sens + 0.5 * spec)
        loss_ref[...] = jnp.reshape(loss, (1, 1))


def kernel(output, target):
    B, num_classes, H, W = output.shape
    assert num_classes == C
    nh = H // TH
    n_total = B * H * W
    loss = pl.pallas_call(
        functools.partial(_loss_kernel, nb=B, nh=nh, n_total=n_total),
        grid=(B, nh),
        in_specs=[
            pl.BlockSpec((1, C, TH, W), lambda b, h: (b, 0, h, 0)),
            pl.BlockSpec((1, TH, W), lambda b, h: (b, h, 0)),
        ],
        out_specs=pl.BlockSpec((1, 1), lambda b, h: (0, 0)),
        out_shape=jax.ShapeDtypeStruct((1, 1), jnp.float32),
        scratch_shapes=[pltpu.VMEM((3, C, 8, 128), jnp.float32)],
    )(output, target)
    return loss[0, 0]
